# baseline (speedup 1.0000x reference)
import functools

import jax
import jax.numpy as jnp
from jax import lax
from jax.experimental import pallas as pl
from jax.experimental.pallas import tpu as pltpu
from jax.experimental.pallas import tpu_sc as plsc

B = 16384
D = 64
DP = 128
NC = 2
NS = 16
NW = NC * NS
B_PER_W = B // NW
CHUNK = 128
N_CHUNKS = B_PER_W // CHUNK


def _make_sc_gather():
    mesh = plsc.VectorSubcoreMesh(core_axis_name="c", subcore_axis_name="s")

    @functools.partial(
        pl.kernel,
        mesh=mesh,
        out_type=jax.ShapeDtypeStruct((B, DP), jnp.float32),
        scratch_types=[
            pltpu.VMEM((B_PER_W,), jnp.int32),
            pltpu.VMEM((CHUNK, DP), jnp.float32),
            pltpu.VMEM((CHUNK, DP), jnp.float32),
            pltpu.SemaphoreType.DMA,
            pltpu.SemaphoreType.DMA,
        ],
    )
    def gather_kernel(idx_hbm, table_hbm, out_hbm, idx_v, rows_a, rows_b, sem_a, sem_b):
        wid = lax.axis_index("s") * NC + lax.axis_index("c")
        base = wid * B_PER_W
        pltpu.sync_copy(idx_hbm.at[pl.ds(base, B_PER_W)], idx_v)
        bufs = [(rows_a, sem_a), (rows_b, sem_b)]
        copies = []
        for j in range(N_CHUNKS):
            rv, sm = bufs[j % 2]
            copies.append(
                pltpu.async_copy(
                    table_hbm.at[idx_v.at[pl.ds(j * CHUNK, CHUNK)]], rv, sm
                )
            )
            if j >= 1:
                copies[j - 1].wait()
                pv, _ = bufs[(j - 1) % 2]
                pltpu.sync_copy(
                    pv, out_hbm.at[pl.ds(base + (j - 1) * CHUNK, CHUNK)]
                )
        copies[N_CHUNKS - 1].wait()
        lv, _ = bufs[(N_CHUNKS - 1) % 2]
        pltpu.sync_copy(
            lv, out_hbm.at[pl.ds(base + (N_CHUNKS - 1) * CHUNK, CHUNK)]
        )

    return gather_kernel


_sc_gather = _make_sc_gather()


def _item_body(x_ref, w_ref, b_ref, o_ref):
    acc = jnp.dot(x_ref[...], w_ref[...], preferred_element_type=jnp.float32)
    o_ref[...] = jnp.maximum(acc + b_ref[...], 0.0)


ROWS_BLK = 1024


def _item_linear(item_x, W_item, b_item):
    return pl.pallas_call(
        _item_body,
        grid=(B // ROWS_BLK,),
        in_specs=[
            pl.BlockSpec((ROWS_BLK, 128), lambda i: (i, 0)),
            pl.BlockSpec((128, D), lambda i: (0, 0)),
            pl.BlockSpec((1, D), lambda i: (0, 0)),
        ],
        out_specs=pl.BlockSpec((ROWS_BLK, D), lambda i: (i, 0)),
        out_shape=jax.ShapeDtypeStruct((B, D), jnp.float32),
    )(item_x, W_item, b_item)


def kernel(user_idx, item_x, emb_table, W_item, b_item):
    idx = user_idx.astype(jnp.int32)
    table2 = emb_table.reshape(-1, DP)
    pairs = _sc_gather(lax.shift_right_logical(idx, 1), table2)
    half = (idx & 1).astype(bool)[:, None]
    hid_user = jnp.where(half, pairs[:, D:], pairs[:, :D])
    hid_item = _item_linear(item_x, W_item, b_item.reshape(1, D))
    return (hid_user, hid_item)


# speedup vs baseline: 2.0791x; 2.0791x over previous
import functools

import jax
import jax.numpy as jnp
from jax import lax
from jax.experimental import pallas as pl
from jax.experimental.pallas import tpu as pltpu
from jax.experimental.pallas import tpu_sc as plsc

B = 16384
D = 64
DP = 128
NC = 2
NS = 16
NW = NC * NS
B_PER_W = B // NW
CHUNK = 128
N_CHUNKS = B_PER_W // CHUNK


def _make_sc_gather():
    mesh = plsc.VectorSubcoreMesh(core_axis_name="c", subcore_axis_name="s")

    @functools.partial(
        pl.kernel,
        mesh=mesh,
        out_type=jax.ShapeDtypeStruct((B, DP), jnp.float32),
        scratch_types=[
            pltpu.VMEM((B_PER_W,), jnp.int32),
            pltpu.VMEM((CHUNK, DP), jnp.float32),
            pltpu.VMEM((CHUNK, DP), jnp.float32),
            pltpu.SemaphoreType.DMA,
            pltpu.SemaphoreType.DMA,
        ],
    )
    def gather_kernel(idx_hbm, table_hbm, out_hbm, idx_v, rows_a, rows_b, sem_a, sem_b):
        wid = lax.axis_index("s") * NC + lax.axis_index("c")
        base = wid * B_PER_W
        pltpu.sync_copy(idx_hbm.at[pl.ds(base, B_PER_W)], idx_v)
        bufs = [(rows_a, sem_a), (rows_b, sem_b)]
        copies = []
        for j in range(N_CHUNKS):
            rv, sm = bufs[j % 2]
            copies.append(
                pltpu.async_copy(
                    table_hbm.at[idx_v.at[pl.ds(j * CHUNK, CHUNK)]], rv, sm
                )
            )
            if j >= 1:
                copies[j - 1].wait()
                pv, _ = bufs[(j - 1) % 2]
                pltpu.sync_copy(
                    pv, out_hbm.at[pl.ds(base + (j - 1) * CHUNK, CHUNK)]
                )
        copies[N_CHUNKS - 1].wait()
        lv, _ = bufs[(N_CHUNKS - 1) % 2]
        pltpu.sync_copy(
            lv, out_hbm.at[pl.ds(base + (N_CHUNKS - 1) * CHUNK, CHUNK)]
        )

    return gather_kernel


_sc_gather = _make_sc_gather()


VB = 8192
N_VBLK = (1000000 + VB - 1) // VB


def _fmt_body(xt_ref, eye_ref, o_ref):
    y = jax.lax.dot_general(
        xt_ref[...], eye_ref[...],
        (((0,), (0,)), ((), ())),
        preferred_element_type=jnp.float32,
    )
    o_ref[:, 0:D] = y


def _format_table(emb_table):
    eye = jnp.eye(D, dtype=jnp.float32)
    return pl.pallas_call(
        _fmt_body,
        grid=(N_VBLK,),
        in_specs=[
            pl.BlockSpec((D, VB), lambda i: (0, i)),
            pl.BlockSpec((D, D), lambda i: (0, 0)),
        ],
        out_specs=pl.BlockSpec((VB, DP), lambda i: (i, 0)),
        out_shape=jax.ShapeDtypeStruct((1000000, DP), jnp.float32),
    )(emb_table.T, eye)


def _item_body(x_ref, w_ref, b_ref, o_ref):
    acc = jnp.dot(x_ref[...], w_ref[...], preferred_element_type=jnp.float32)
    o_ref[...] = jnp.maximum(acc + b_ref[...], 0.0)


ROWS_BLK = 1024


def _item_linear(item_x, W_item, b_item):
    return pl.pallas_call(
        _item_body,
        grid=(B // ROWS_BLK,),
        in_specs=[
            pl.BlockSpec((ROWS_BLK, 128), lambda i: (i, 0)),
            pl.BlockSpec((128, D), lambda i: (0, 0)),
            pl.BlockSpec((1, D), lambda i: (0, 0)),
        ],
        out_specs=pl.BlockSpec((ROWS_BLK, D), lambda i: (i, 0)),
        out_shape=jax.ShapeDtypeStruct((B, D), jnp.float32),
    )(item_x, W_item, b_item)


def kernel(user_idx, item_x, emb_table, W_item, b_item):
    idx = user_idx.astype(jnp.int32)
    table2 = _format_table(emb_table)
    rows = _sc_gather(idx, table2)
    hid_user = rows[:, :D]
    hid_item = _item_linear(item_x, W_item, b_item.reshape(1, D))
    return (hid_user, hid_item)
